# baseline (speedup 1.0000x reference)
import functools

import jax
import jax.numpy as jnp
from jax import lax
from jax.experimental import pallas as pl
from jax.experimental.pallas import tpu as pltpu
from jax.experimental.pallas import tpu_sc as plsc

_VOCAB = 1000000
_D = 16
_NUM_COLS = 5
_B = 16384
_NC, _NS = 2, 16
_NW = _NC * _NS
_BPW = _B // _NW
_CPW = _BPW * _NUM_COLS
_CHUNK = 128


def _sc_call(tgt_idx, ctx_idx, tgt_tab, ctx_tab):
    mesh = plsc.VectorSubcoreMesh(core_axis_name="c", subcore_axis_name="s")

    @functools.partial(
        pl.kernel,
        mesh=mesh,
        compiler_params=pltpu.CompilerParams(
            needs_layout_passes=False, use_tc_tiling_on_sc=False),
        out_type=jax.ShapeDtypeStruct((_B * _NUM_COLS,), jnp.float32),
        scratch_types=[
            pltpu.VMEM((_BPW,), jnp.int32),
            pltpu.VMEM((_CPW,), jnp.int32),
            pltpu.VMEM((_BPW, _D), jnp.float32),
            pltpu.VMEM((_CPW, _D), jnp.float32),
            pltpu.VMEM((_CPW,), jnp.float32),
            pltpu.SemaphoreType.DMA,
        ],
    )
    def body(tgt_idx_hbm, ctx_idx_hbm, tgt_tab_hbm, ctx_tab_hbm, out_hbm,
             tidx_v, cidx_v, trows_v, crows_v, out_v, sem):
        wid = lax.axis_index("s") * _NC + lax.axis_index("c")
        base = wid * _BPW
        cbase = wid * _CPW

        pltpu.sync_copy(tgt_idx_hbm.at[pl.ds(base, _BPW)], tidx_v)
        pltpu.sync_copy(ctx_idx_hbm.at[pl.ds(cbase, _CPW)], cidx_v)

        copies = []
        for j in range(_BPW // _CHUNK):
            s = pl.ds(j * _CHUNK, _CHUNK)
            copies.append(pltpu.async_copy(
                tgt_tab_hbm.at[tidx_v.at[s]], trows_v.at[s], sem))
        for j in range(_CPW // _CHUNK):
            s = pl.ds(j * _CHUNK, _CHUNK)
            copies.append(pltpu.async_copy(
                ctx_tab_hbm.at[cidx_v.at[s]], crows_v.at[s], sem))
        for c in copies:
            c.wait()

        lane = lax.iota(jnp.int32, 16)
        last = lane == 15

        def step(b, carry):
            tvec = trows_v[b]
            b5 = b * _NUM_COLS
            for c in range(_NUM_COLS):
                prod = crows_v[b5 + c] * tvec
                s = plsc.cumsum(prod)
                idx = jnp.zeros((16,), jnp.int32) + (b5 + c)
                plsc.store_scatter(out_v, [idx], s, mask=last)
            return carry

        lax.fori_loop(0, _BPW, step, 0)

        pltpu.sync_copy(out_v, out_hbm.at[pl.ds(cbase, _CPW)])

    return body(tgt_idx, ctx_idx, tgt_tab, ctx_tab)


def kernel(target, context, target_table, context_table):
    out = _sc_call(target.reshape(-1), context.reshape(-1),
                   target_table, context_table)
    return out.reshape(_B, _NUM_COLS)


# speedup vs baseline: 1.3789x; 1.3789x over previous
import functools

import jax
import jax.numpy as jnp
from jax import lax
from jax.experimental import pallas as pl
from jax.experimental.pallas import tpu as pltpu
from jax.experimental.pallas import tpu_sc as plsc

_VOCAB = 1000000
_D = 16
_NUM_COLS = 5
_B = 16384
_NC, _NS = 2, 16
_NW = _NC * _NS
_BPW = _B // _NW
_CPW = _BPW * _NUM_COLS
_CBB = 16
_CCHUNK = _CBB * _NUM_COLS
_NCHUNK = _BPW // _CBB


def _sc_call(tgt_idx, ctx_idx, tgt_tab, ctx_tab):
    mesh = plsc.VectorSubcoreMesh(core_axis_name="c", subcore_axis_name="s")

    @functools.partial(
        pl.kernel,
        mesh=mesh,
        compiler_params=pltpu.CompilerParams(
            needs_layout_passes=False, use_tc_tiling_on_sc=True),
        out_type=jax.ShapeDtypeStruct((_B * _NUM_COLS,), jnp.float32),
        scratch_types=[
            pltpu.VMEM((_BPW,), jnp.int32),
            pltpu.VMEM((_CPW,), jnp.int32),
            pltpu.VMEM((_BPW, _D), jnp.float32),
            pltpu.VMEM((_CCHUNK, _D), jnp.float32),
            pltpu.VMEM((_CCHUNK, _D), jnp.float32),
            pltpu.VMEM((_CPW,), jnp.float32),
            pltpu.SemaphoreType.DMA,
            pltpu.SemaphoreType.DMA,
            pltpu.SemaphoreType.DMA,
        ],
    )
    def body(tgt_idx_hbm, ctx_idx_hbm, tgt_tab_hbm, ctx_tab_hbm, out_hbm,
             tidx_v, cidx_v, trows_v, ca_v, cb_v, out_v, tsem, asem, bsem):
        wid = lax.axis_index("s") * _NC + lax.axis_index("c")
        base = wid * _BPW
        cbase = wid * _CPW

        pltpu.sync_copy(tgt_idx_hbm.at[pl.ds(base, _BPW)], tidx_v)
        pltpu.sync_copy(ctx_idx_hbm.at[pl.ds(cbase, _CPW)], cidx_v)

        def fire_t(g, carry):
            vec = tidx_v[pl.ds(g * _D, _D)]
            for k in range(_D):
                pltpu.async_copy(tgt_tab_hbm.at[vec[k]],
                                 trows_v.at[g * _D + k], tsem)
            return carry

        lax.fori_loop(0, _BPW // _D, fire_t, 0)

        def fire(chunk, buf, sem):
            for j in range(_CCHUNK // _D):
                vec = cidx_v[pl.ds(chunk * _CCHUNK + j * _D, _D)]
                for k in range(_D):
                    pltpu.async_copy(ctx_tab_hbm.at[vec[k]],
                                     buf.at[j * _D + k], sem)

        def drain(buf, sem):
            pltpu.make_async_copy(
                ctx_tab_hbm.at[pl.ds(0, _CCHUNK)], buf, sem).wait()

        fire(0, ca_v, asem)
        fire(1, cb_v, bsem)
        pltpu.make_async_copy(
            tgt_tab_hbm.at[pl.ds(0, _BPW)], trows_v, tsem).wait()

        lane = lax.iota(jnp.int32, 16)
        last = lane == 15

        def compute(chunk, buf):
            b0 = chunk * _CBB
            for bb in range(_CBB):
                tvec = trows_v[b0 + bb]
                r0 = bb * _NUM_COLS
                o0 = (b0 + bb) * _NUM_COLS
                for c in range(_NUM_COLS):
                    prod = buf[r0 + c] * tvec
                    s = plsc.cumsum(prod)
                    idx = jnp.zeros((16,), jnp.int32) + (o0 + c)
                    plsc.store_scatter(out_v, [idx], s, mask=last)

        def step(k, carry):
            drain(ca_v, asem)
            compute(2 * k, ca_v)
            fire(2 * k + 2, ca_v, asem)
            drain(cb_v, bsem)
            compute(2 * k + 1, cb_v)
            fire(2 * k + 3, cb_v, bsem)
            return carry

        lax.fori_loop(0, _NCHUNK // 2 - 1, step, 0)

        drain(ca_v, asem)
        compute(_NCHUNK - 2, ca_v)
        drain(cb_v, bsem)
        compute(_NCHUNK - 1, cb_v)

        pltpu.sync_copy(out_v, out_hbm.at[pl.ds(cbase, _CPW)])

    return body(tgt_idx, ctx_idx, tgt_tab, ctx_tab)


def kernel(target, context, target_table, context_table):
    out = _sc_call(target.reshape(-1), context.reshape(-1),
                   target_table, context_table)
    return out.reshape(_B, _NUM_COLS)
